# Initial kernel scaffold; baseline (speedup 1.0000x reference)
import jax
import jax.numpy as jnp
from jax.experimental import pallas as pl


def kernel(x, sadj, fadj, W_s1, b_s1, W_s2, b_s2, W_f1, b_f1, W_f2, b_f2, W_c1, b_c1, W_c2, b_c2, A1, ab1, A2, W_mlp, b_mlp, c_f, c_s, c_com, W_dec, b_dec, bn_g, bn_b, W_pi, b_pi, W_disp, b_disp, W_mean, b_mean):
    raise NotImplementedError("write your pallas kernel here")


# speedup vs baseline: 5.0478x; 5.0478x over previous
import functools

import jax
import jax.numpy as jnp
from jax import lax
from jax.experimental import pallas as pl
from jax.experimental.pallas import tpu as pltpu
from jax.experimental.pallas import tpu_sc as plsc

def _dot(a, b, preferred_element_type=jnp.float32):
  return jnp.dot(a, b, preferred_element_type=preferred_element_type)

N = 10000
E = 320000
NS = 16
EPT = E // NS
WIN = 80
NWIN = EPT // WIN
ZBLK = 40
NZB = N // ZBLK
ZITER = (NZB + NS - 1) // NS


def _make_agg(F):
  mesh = plsc.VectorSubcoreMesh(core_axis_name="c", subcore_axis_name="s",
                                num_cores=2, num_subcores=NS)

  @functools.partial(
      pl.kernel,
      out_type=(jax.ShapeDtypeStruct((N, F), jnp.float32),
                jax.ShapeDtypeStruct((N, F), jnp.float32)),
      mesh=mesh,
      scratch_types=[
          pltpu.VMEM((WIN,), jnp.int32),
          pltpu.VMEM((WIN,), jnp.int32),
          pltpu.VMEM((WIN, F), jnp.float32),
          pltpu.VMEM_SHARED((N, F), jnp.float32),
          pltpu.SemaphoreType.DMA,
      ],
  )
  def agg(hs, hf, s_src, s_dst, f_src, f_dst, zeros_hbm, out_s, out_f,
          src_v, dst_v, rows_v, acc, sem):
    c = lax.axis_index("c")
    s = lax.axis_index("s")

    def zinit(j, carry):
      b = j * NS + s

      @pl.when(b < NZB)
      def _():
        pltpu.sync_copy(zeros_hbm.at[pl.ds(b * ZBLK, ZBLK)],
                        acc.at[pl.ds(b * ZBLK, ZBLK)])

      return carry

    lax.fori_loop(0, ZITER, zinit, 0)
    plsc.subcore_barrier()

    def run(src_hbm, dst_hbm, tbl_hbm, out_hbm):
      base = s * EPT

      def body(w, carry):
        off = base + w * WIN
        pltpu.sync_copy(src_hbm.at[pl.ds(off, WIN)], src_v)
        pltpu.sync_copy(dst_hbm.at[pl.ds(off, WIN)], dst_v)
        pltpu.async_copy(tbl_hbm.at[src_v], rows_v, sem).wait()
        pltpu.sync_copy(rows_v, acc.at[dst_v], add=True)
        return carry

      lax.fori_loop(0, NWIN, body, 0)
      plsc.subcore_barrier()

      def zout(j, carry):
        b = j * NS + s

        @pl.when(b < NZB)
        def _():
          pltpu.sync_copy(acc.at[pl.ds(b * ZBLK, ZBLK)],
                          out_hbm.at[pl.ds(b * ZBLK, ZBLK)])

        return carry

      lax.fori_loop(0, ZITER, zout, 0)

    @pl.when(c == 0)
    def _():
      run(s_src, s_dst, hs, out_s)

    @pl.when(c == 1)
    def _():
      run(f_src, f_dst, hf, out_f)

  return agg


@functools.lru_cache(maxsize=None)
def _get_agg(F):
  return _make_agg(F)


BN = 2000


def _full(shape):
  return pl.BlockSpec(shape, lambda i: (0, 0))


def _rows(cols):
  return pl.BlockSpec((BN, cols), lambda i: (i, 0))


def _k1_body(x_ref, wa_ref, wb_ref, oa_ref, ob_ref):
  xb = x_ref[...]
  oa_ref[...] = _dot(xb, wa_ref[...], preferred_element_type=jnp.float32)
  ob_ref[...] = _dot(xb, wb_ref[...], preferred_element_type=jnp.float32)


def _k1(x, wa, wb):
  return pl.pallas_call(
      _k1_body,
      grid=(N // BN,),
      in_specs=[_rows(128), _full((128, 128)), _full((128, 128))],
      out_specs=(_rows(128), _rows(128)),
      out_shape=(jax.ShapeDtypeStruct((N, 128), jnp.float32),
                 jax.ShapeDtypeStruct((N, 128), jnp.float32)),
  )(x, wa, wb)


def _k2_body(a_ref, b_ref, ba_ref, bb_ref, ws_ref, wf_ref, o_ref):
  ha = jnp.maximum(a_ref[...] + ba_ref[...], 0.0)
  hb = jnp.maximum(b_ref[...] + bb_ref[...], 0.0)
  gs = _dot(ha, ws_ref[...], preferred_element_type=jnp.float32)
  gf = _dot(hb, wf_ref[...], preferred_element_type=jnp.float32)
  o_ref[...] = jnp.concatenate([gs, gf], axis=1)


def _k2(As, Af, b1s, b1f, wblk_s, wblk_f):
  return pl.pallas_call(
      _k2_body,
      grid=(N // BN,),
      in_specs=[_rows(128), _rows(128), _full((1, 128)), _full((1, 128)),
                _full((128, 64)), _full((128, 64))],
      out_specs=_rows(128),
      out_shape=jax.ShapeDtypeStruct((N, 128), jnp.float32),
  )(As, Af, b1s, b1f, wblk_s, wblk_f)


def _k3_body(bs_ref, bf_ref,
             b_s2_ref, b_c2_ref, b_f2_ref,
             a1_ref, ab1_ref, a2_ref, cvec_ref, wmlp_ref, bmlp_ref,
             wdec_ref, bdec_ref,
             com1_ref, com2_ref, emb_ref, hpre_ref, stats_ref):
  i = pl.program_id(0)
  bs = bs_ref[...][:, :64]
  bf = bf_ref[...][:, 64:]
  emb1 = bs[:, :32] + b_s2_ref[...]
  com1 = bs[:, 32:] + b_c2_ref[...]
  com2 = bf[:, :32] + b_c2_ref[...]
  emb2 = bf[:, 32:] + b_f2_ref[...]
  com1_ref[...] = com1
  com2_ref[...] = com2
  comavg = (com1 + com2) * 0.5

  a1 = a1_ref[...]
  ab1 = ab1_ref[...]
  a2 = a2_ref[...]

  def att(zi):
    t = jnp.tanh(_dot(zi, a1, preferred_element_type=jnp.float32) + ab1)
    return _dot(t, a2, preferred_element_type=jnp.float32)

  w1 = att(emb1)
  w2 = att(comavg)
  w3 = att(emb2)
  m = jnp.maximum(jnp.maximum(w1, w2), w3)
  e1 = jnp.exp(w1 - m)
  e2 = jnp.exp(w2 - m)
  e3 = jnp.exp(w3 - m)
  inv = 1.0 / (e1 + e2 + e3)
  emb_att = (e1 * emb1 + e2 * comavg + e3 * emb2) * inv

  cv = jnp.tanh(cvec_ref[...])
  emb1c = cv[0, 0] * emb1 + cv[0, 1] * emb2 + cv[0, 2] * comavg

  emb = _dot(emb1c + emb_att, wmlp_ref[...],
                preferred_element_type=jnp.float32) + bmlp_ref[...]
  emb_ref[...] = emb

  hpre = _dot(emb, wdec_ref[...],
                 preferred_element_type=jnp.float32) + bdec_ref[...]
  hpre_ref[...] = hpre

  s1 = jnp.sum(hpre, axis=0, keepdims=True)
  s2 = jnp.sum(hpre * hpre, axis=0, keepdims=True)
  block = jnp.concatenate([s1, s2, jnp.zeros((6, 64), jnp.float32)], axis=0)

  @pl.when(i == 0)
  def _():
    stats_ref[...] = jnp.zeros_like(stats_ref)

  stats_ref[...] += block


def _k3(AggS, AggF, b_s2, b_c2, b_f2, A1, ab1, a2col, cvec,
        W_mlp, b_mlp, W_dec, b_dec):
  return pl.pallas_call(
      _k3_body,
      grid=(N // BN,),
      in_specs=[_rows(128), _rows(128),
                _full((1, 32)), _full((1, 32)),
                _full((1, 32)), _full((32, 16)), _full((1, 16)),
                _full((16, 1)), _full((1, 3)), _full((32, 32)),
                _full((1, 32)), _full((32, 64)), _full((1, 64))],
      out_specs=(_rows(32), _rows(32), _rows(32), _rows(64),
                 pl.BlockSpec((8, 64), lambda i: (0, 0))),
      out_shape=(jax.ShapeDtypeStruct((N, 32), jnp.float32),
                 jax.ShapeDtypeStruct((N, 32), jnp.float32),
                 jax.ShapeDtypeStruct((N, 32), jnp.float32),
                 jax.ShapeDtypeStruct((N, 64), jnp.float32),
                 jax.ShapeDtypeStruct((8, 64), jnp.float32)),
  )(AggS, AggF, b_s2, b_c2, b_f2, A1, ab1, a2col, cvec,
    W_mlp, b_mlp, W_dec, b_dec)


def _k4_body(hpre_ref, stats_ref, bng_ref, bnb_ref,
             wpi_ref, bpi_ref, wdisp_ref, bdisp_ref, wmean_ref, bmean_ref,
             pi_ref, disp_ref, mean_ref):
  stats = stats_ref[...]
  mu = stats[0:1, :] * (1.0 / N)
  ex2 = stats[1:2, :] * (1.0 / N)
  var = ex2 - mu * mu
  scale = bng_ref[...] * jax.lax.rsqrt(var + 1e-5)
  h = (hpre_ref[...] - mu) * scale + bnb_ref[...]
  h = jnp.maximum(h, 0.0)
  pi_ref[...] = jax.nn.sigmoid(
      _dot(h, wpi_ref[...], preferred_element_type=jnp.float32)
      + bpi_ref[...])
  disp_ref[...] = jnp.clip(
      jax.nn.softplus(_dot(h, wdisp_ref[...],
                              preferred_element_type=jnp.float32)
                      + bdisp_ref[...]), 1e-4, 1e4)
  mean_ref[...] = jnp.clip(
      jnp.exp(_dot(h, wmean_ref[...], preferred_element_type=jnp.float32)
              + bmean_ref[...]), 1e-5, 1e6)


def _k4(hpre, stats, bn_g, bn_b, W_pi, b_pi, W_disp, b_disp, W_mean, b_mean):
  return pl.pallas_call(
      _k4_body,
      grid=(N // BN,),
      in_specs=[_rows(64), pl.BlockSpec((8, 64), lambda i: (0, 0)),
                _full((1, 64)), _full((1, 64)),
                _full((64, 128)), _full((1, 128)),
                _full((64, 128)), _full((1, 128)),
                _full((64, 128)), _full((1, 128))],
      out_specs=(_rows(128), _rows(128), _rows(128)),
      out_shape=(jax.ShapeDtypeStruct((N, 128), jnp.float32),
                 jax.ShapeDtypeStruct((N, 128), jnp.float32),
                 jax.ShapeDtypeStruct((N, 128), jnp.float32)),
  )(hpre, stats, bn_g, bn_b, W_pi, b_pi, W_disp, b_disp, W_mean, b_mean)


def kernel(x, sadj, fadj, W_s1, b_s1, W_s2, b_s2, W_f1, b_f1, W_f2, b_f2,
           W_c1, b_c1, W_c2, b_c2, A1, ab1, A2, W_mlp, b_mlp, c_f, c_s,
           c_com, W_dec, b_dec, bn_g, bn_b, W_pi, b_pi, W_disp, b_disp,
           W_mean, b_mean):
  f32 = jnp.float32
  Wsc = jnp.concatenate([W_s1, W_c1], axis=1)
  Wcf = jnp.concatenate([W_c1, W_f1], axis=1)
  b1s = jnp.concatenate([b_s1, b_c1])[None, :]
  b1f = jnp.concatenate([b_c1, b_f1])[None, :]
  Z = jnp.zeros((64, 32), f32)
  wblk_s = jnp.concatenate(
      [jnp.concatenate([W_s2, Z], axis=1),
       jnp.concatenate([Z, W_c2], axis=1)], axis=0)
  wblk_f = jnp.concatenate(
      [jnp.concatenate([W_c2, Z], axis=1),
       jnp.concatenate([Z, W_f2], axis=1)], axis=0)
  cvec = jnp.stack([c_f, c_s, c_com]).reshape(1, 3)
  s_src, s_dst = sadj[0], sadj[1]
  f_src, f_dst = fadj[0], fadj[1]
  zeros128 = jnp.zeros((N, 128), f32)

  hs, hf = _k1(x, Wsc, Wcf)
  As, Af = _get_agg(128)(hs, hf, s_src, s_dst, f_src, f_dst, zeros128)

  g_packed = _k2(As, Af, b1s, b1f, wblk_s, wblk_f)
  AggS, AggF = _get_agg(128)(g_packed, g_packed, s_src, s_dst, f_src,
                             f_dst, zeros128)

  com1, com2, emb, hpre, stats = _k3(
      AggS, AggF, b_s2[None, :], b_c2[None, :],
      b_f2[None, :], A1, ab1[None, :], A2, cvec, W_mlp,
      b_mlp[None, :], W_dec, b_dec[None, :])
  pi, disp, mean = _k4(hpre, stats, bn_g[None, :], bn_b[None, :],
                       W_pi, b_pi[None, :], W_disp, b_disp[None, :],
                       W_mean, b_mean[None, :])
  return (com1, com2, emb, pi, disp, mean)


# speedup vs baseline: 8.9897x; 1.7809x over previous
import functools

import jax
import jax.numpy as jnp
from jax import lax
from jax.experimental import pallas as pl
from jax.experimental.pallas import tpu as pltpu
from jax.experimental.pallas import tpu_sc as plsc

def _dot(a, b, preferred_element_type=jnp.float32):
  return jnp.dot(a, b, preferred_element_type=preferred_element_type)

N = 10000
E = 320000
NS = 16
WIN = 80
NROW = E // WIN
WROWS = NROW // NS
ZBLK = 40
NZB = N // ZBLK
ZITER = (NZB + NS - 1) // NS


def _make_agg(F):
  mesh = plsc.VectorSubcoreMesh(core_axis_name="c", subcore_axis_name="s",
                                num_cores=2, num_subcores=NS)

  @functools.partial(
      pl.kernel,
      out_type=(jax.ShapeDtypeStruct((N, F), jnp.float32),
                jax.ShapeDtypeStruct((N, F), jnp.float32)),
      mesh=mesh,
      scratch_types=[
          pltpu.VMEM((4, 2, WIN), jnp.int32),
          pltpu.VMEM((2, WIN, F), jnp.float32),
          pltpu.VMEM_SHARED((N, F), jnp.float32),
          pltpu.SemaphoreType.DMA,
          pltpu.SemaphoreType.DMA,
          pltpu.SemaphoreType.DMA,
          pltpu.SemaphoreType.DMA,
          pltpu.SemaphoreType.DMA,
          pltpu.SemaphoreType.DMA,
          pltpu.SemaphoreType.DMA,
      ],
  )
  def agg(hs, hf, ep_s, ep_f, zeros_hbm, out_s, out_f,
          idx, rows, acc, gsem, ssem0, ssem1, is0, is1, is2, is3):
    c = lax.axis_index("c")
    s = lax.axis_index("s")
    ssems = (ssem0, ssem1)
    isems = (is0, is1, is2, is3)

    def zinit(j, carry):
      b = j * NS + s

      @pl.when(b < NZB)
      def _():
        pltpu.sync_copy(zeros_hbm.at[pl.ds(b * ZBLK, ZBLK)],
                        acc.at[pl.ds(b * ZBLK, ZBLK)])

      return carry

    lax.fori_loop(0, ZITER, zinit, 0)
    plsc.subcore_barrier()

    def run(ep_hbm, tbl_hbm, out_hbm):
      base = s * WROWS

      def fetch(w, q):
        pltpu.async_copy(ep_hbm.at[base + w], idx.at[q], isems[q])

      def wait_i(q):
        pltpu.make_async_copy(ep_hbm.at[base], idx.at[q], isems[q]).wait()

      def gath(w, q, rb):
        del w
        pltpu.async_copy(tbl_hbm.at[idx.at[q, 0]], rows.at[rb], gsem)

      def wait_g(rb):
        pltpu.make_async_copy(tbl_hbm.at[idx.at[0, 0]], rows.at[rb],
                              gsem).wait()

      def scat(w, q, rb):
        del w
        pltpu.async_copy(rows.at[rb], acc.at[idx.at[q, 1]], ssems[rb],
                         add=True)

      def wait_s(rb):
        pltpu.make_async_copy(rows.at[rb], acc.at[idx.at[0, 1]],
                              ssems[rb]).wait()

      fetch(0, 0)
      fetch(1, 1)
      fetch(2, 2)
      wait_i(0)
      gath(0, 0, 0)
      wait_g(0)
      scat(0, 0, 0)
      fetch(3, 3)
      wait_i(1)
      gath(1, 1, 1)
      wait_g(1)
      scat(1, 1, 1)
      wait_s(0)
      fetch(4, 0)
      wait_i(2)
      gath(2, 2, 0)

      def batch(kb, carry):
        for j in range(8):
          w = kb * 8 + j + 2
          rb = j % 2
          q = (j + 2) % 4
          wait_g(rb)
          scat(w, q, rb)
          wait_s(1 - rb)
          @pl.when(w + 3 < WROWS)
          def _():
            fetch(w + 3, (q + 3) % 4)
          @pl.when(w + 1 < WROWS)
          def _():
            wait_i((q + 1) % 4)
            gath(w + 1, (q + 1) % 4, 1 - rb)
        return carry

      lax.fori_loop(0, (WROWS - 2) // 8, batch, 0)
      wait_s(1)
      plsc.subcore_barrier()

      def zout(j, carry):
        b = j * NS + s

        @pl.when(b < NZB)
        def _():
          pltpu.sync_copy(acc.at[pl.ds(b * ZBLK, ZBLK)],
                          out_hbm.at[pl.ds(b * ZBLK, ZBLK)])

        return carry

      lax.fori_loop(0, ZITER, zout, 0)

    @pl.when(c == 0)
    def _():
      run(ep_s, hs, out_s)

    @pl.when(c == 1)
    def _():
      run(ep_f, hf, out_f)

  return agg


@functools.lru_cache(maxsize=None)
def _get_agg(F):
  return _make_agg(F)


BN = 2000


def _full(shape):
  return pl.BlockSpec(shape, lambda i: (0, 0))


def _rows(cols):
  return pl.BlockSpec((BN, cols), lambda i: (i, 0))


def _k1_body(x_ref, wa_ref, wb_ref, oa_ref, ob_ref):
  xb = x_ref[...]
  oa_ref[...] = _dot(xb, wa_ref[...], preferred_element_type=jnp.float32)
  ob_ref[...] = _dot(xb, wb_ref[...], preferred_element_type=jnp.float32)


def _k1(x, wa, wb):
  return pl.pallas_call(
      _k1_body,
      grid=(N // BN,),
      in_specs=[_rows(128), _full((128, 128)), _full((128, 128))],
      out_specs=(_rows(128), _rows(128)),
      out_shape=(jax.ShapeDtypeStruct((N, 128), jnp.float32),
                 jax.ShapeDtypeStruct((N, 128), jnp.float32)),
  )(x, wa, wb)


def _k2_body(a_ref, b_ref, ba_ref, bb_ref, ws_ref, wf_ref, o_ref):
  ha = jnp.maximum(a_ref[...] + ba_ref[...], 0.0)
  hb = jnp.maximum(b_ref[...] + bb_ref[...], 0.0)
  gs = _dot(ha, ws_ref[...], preferred_element_type=jnp.float32)
  gf = _dot(hb, wf_ref[...], preferred_element_type=jnp.float32)
  o_ref[...] = jnp.concatenate([gs, gf], axis=1)


def _k2(As, Af, b1s, b1f, wblk_s, wblk_f):
  return pl.pallas_call(
      _k2_body,
      grid=(N // BN,),
      in_specs=[_rows(128), _rows(128), _full((1, 128)), _full((1, 128)),
                _full((128, 64)), _full((128, 64))],
      out_specs=_rows(128),
      out_shape=jax.ShapeDtypeStruct((N, 128), jnp.float32),
  )(As, Af, b1s, b1f, wblk_s, wblk_f)


def _k3_body(bs_ref, bf_ref,
             b_s2_ref, b_c2_ref, b_f2_ref,
             a1_ref, ab1_ref, a2_ref, cvec_ref, wmlp_ref, bmlp_ref,
             wdec_ref, bdec_ref,
             com1_ref, com2_ref, emb_ref, hpre_ref, stats_ref):
  i = pl.program_id(0)
  bs = bs_ref[...][:, :64]
  bf = bf_ref[...][:, 64:]
  emb1 = bs[:, :32] + b_s2_ref[...]
  com1 = bs[:, 32:] + b_c2_ref[...]
  com2 = bf[:, :32] + b_c2_ref[...]
  emb2 = bf[:, 32:] + b_f2_ref[...]
  com1_ref[...] = com1
  com2_ref[...] = com2
  comavg = (com1 + com2) * 0.5

  a1 = a1_ref[...]
  ab1 = ab1_ref[...]
  a2 = a2_ref[...]

  def att(zi):
    t = jnp.tanh(_dot(zi, a1, preferred_element_type=jnp.float32) + ab1)
    return _dot(t, a2, preferred_element_type=jnp.float32)

  w1 = att(emb1)
  w2 = att(comavg)
  w3 = att(emb2)
  m = jnp.maximum(jnp.maximum(w1, w2), w3)
  e1 = jnp.exp(w1 - m)
  e2 = jnp.exp(w2 - m)
  e3 = jnp.exp(w3 - m)
  inv = 1.0 / (e1 + e2 + e3)
  emb_att = (e1 * emb1 + e2 * comavg + e3 * emb2) * inv

  cv = jnp.tanh(cvec_ref[...])
  emb1c = cv[0, 0] * emb1 + cv[0, 1] * emb2 + cv[0, 2] * comavg

  emb = _dot(emb1c + emb_att, wmlp_ref[...],
                preferred_element_type=jnp.float32) + bmlp_ref[...]
  emb_ref[...] = emb

  hpre = _dot(emb, wdec_ref[...],
                 preferred_element_type=jnp.float32) + bdec_ref[...]
  hpre_ref[...] = hpre

  s1 = jnp.sum(hpre, axis=0, keepdims=True)
  s2 = jnp.sum(hpre * hpre, axis=0, keepdims=True)
  block = jnp.concatenate([s1, s2, jnp.zeros((6, 64), jnp.float32)], axis=0)

  @pl.when(i == 0)
  def _():
    stats_ref[...] = jnp.zeros_like(stats_ref)

  stats_ref[...] += block


def _k3(AggS, AggF, b_s2, b_c2, b_f2, A1, ab1, a2col, cvec,
        W_mlp, b_mlp, W_dec, b_dec):
  return pl.pallas_call(
      _k3_body,
      grid=(N // BN,),
      in_specs=[_rows(128), _rows(128),
                _full((1, 32)), _full((1, 32)),
                _full((1, 32)), _full((32, 16)), _full((1, 16)),
                _full((16, 1)), _full((1, 3)), _full((32, 32)),
                _full((1, 32)), _full((32, 64)), _full((1, 64))],
      out_specs=(_rows(32), _rows(32), _rows(32), _rows(64),
                 pl.BlockSpec((8, 64), lambda i: (0, 0))),
      out_shape=(jax.ShapeDtypeStruct((N, 32), jnp.float32),
                 jax.ShapeDtypeStruct((N, 32), jnp.float32),
                 jax.ShapeDtypeStruct((N, 32), jnp.float32),
                 jax.ShapeDtypeStruct((N, 64), jnp.float32),
                 jax.ShapeDtypeStruct((8, 64), jnp.float32)),
  )(AggS, AggF, b_s2, b_c2, b_f2, A1, ab1, a2col, cvec,
    W_mlp, b_mlp, W_dec, b_dec)


def _k4_body(hpre_ref, stats_ref, bng_ref, bnb_ref,
             wpi_ref, bpi_ref, wdisp_ref, bdisp_ref, wmean_ref, bmean_ref,
             pi_ref, disp_ref, mean_ref):
  stats = stats_ref[...]
  mu = stats[0:1, :] * (1.0 / N)
  ex2 = stats[1:2, :] * (1.0 / N)
  var = ex2 - mu * mu
  scale = bng_ref[...] * jax.lax.rsqrt(var + 1e-5)
  h = (hpre_ref[...] - mu) * scale + bnb_ref[...]
  h = jnp.maximum(h, 0.0)
  pi_ref[...] = jax.nn.sigmoid(
      _dot(h, wpi_ref[...], preferred_element_type=jnp.float32)
      + bpi_ref[...])
  disp_ref[...] = jnp.clip(
      jax.nn.softplus(_dot(h, wdisp_ref[...],
                              preferred_element_type=jnp.float32)
                      + bdisp_ref[...]), 1e-4, 1e4)
  mean_ref[...] = jnp.clip(
      jnp.exp(_dot(h, wmean_ref[...], preferred_element_type=jnp.float32)
              + bmean_ref[...]), 1e-5, 1e6)


def _k4(hpre, stats, bn_g, bn_b, W_pi, b_pi, W_disp, b_disp, W_mean, b_mean):
  return pl.pallas_call(
      _k4_body,
      grid=(N // BN,),
      in_specs=[_rows(64), pl.BlockSpec((8, 64), lambda i: (0, 0)),
                _full((1, 64)), _full((1, 64)),
                _full((64, 128)), _full((1, 128)),
                _full((64, 128)), _full((1, 128)),
                _full((64, 128)), _full((1, 128))],
      out_specs=(_rows(128), _rows(128), _rows(128)),
      out_shape=(jax.ShapeDtypeStruct((N, 128), jnp.float32),
                 jax.ShapeDtypeStruct((N, 128), jnp.float32),
                 jax.ShapeDtypeStruct((N, 128), jnp.float32)),
  )(hpre, stats, bn_g, bn_b, W_pi, b_pi, W_disp, b_disp, W_mean, b_mean)


def kernel(x, sadj, fadj, W_s1, b_s1, W_s2, b_s2, W_f1, b_f1, W_f2, b_f2,
           W_c1, b_c1, W_c2, b_c2, A1, ab1, A2, W_mlp, b_mlp, c_f, c_s,
           c_com, W_dec, b_dec, bn_g, bn_b, W_pi, b_pi, W_disp, b_disp,
           W_mean, b_mean):
  f32 = jnp.float32
  Wsc = jnp.concatenate([W_s1, W_c1], axis=1)
  Wcf = jnp.concatenate([W_c1, W_f1], axis=1)
  b1s = jnp.concatenate([b_s1, b_c1])[None, :]
  b1f = jnp.concatenate([b_c1, b_f1])[None, :]
  Z = jnp.zeros((64, 32), f32)
  wblk_s = jnp.concatenate(
      [jnp.concatenate([W_s2, Z], axis=1),
       jnp.concatenate([Z, W_c2], axis=1)], axis=0)
  wblk_f = jnp.concatenate(
      [jnp.concatenate([W_c2, Z], axis=1),
       jnp.concatenate([Z, W_f2], axis=1)], axis=0)
  cvec = jnp.stack([c_f, c_s, c_com]).reshape(1, 3)
  ep_s = jnp.stack([sadj[0].reshape(NROW, WIN),
                    sadj[1].reshape(NROW, WIN)], axis=1)
  ep_f = jnp.stack([fadj[0].reshape(NROW, WIN),
                    fadj[1].reshape(NROW, WIN)], axis=1)
  zeros128 = jnp.zeros((N, 128), f32)

  hs, hf = _k1(x, Wsc, Wcf)
  As, Af = _get_agg(128)(hs, hf, ep_s, ep_f, zeros128)

  g_packed = _k2(As, Af, b1s, b1f, wblk_s, wblk_f)
  AggS, AggF = _get_agg(128)(g_packed, g_packed, ep_s, ep_f, zeros128)

  com1, com2, emb, hpre, stats = _k3(
      AggS, AggF, b_s2[None, :], b_c2[None, :],
      b_f2[None, :], A1, ab1[None, :], A2, cvec, W_mlp,
      b_mlp[None, :], W_dec, b_dec[None, :])
  pi, disp, mean = _k4(hpre, stats, bn_g[None, :], bn_b[None, :],
                       W_pi, b_pi[None, :], W_disp, b_disp[None, :],
                       W_mean, b_mean[None, :])
  return (com1, com2, emb, pi, disp, mean)


# speedup vs baseline: 14.3798x; 1.5996x over previous
import functools

import jax
import jax.numpy as jnp
from jax import lax
from jax.experimental import pallas as pl
from jax.experimental.pallas import tpu as pltpu
from jax.experimental.pallas import tpu_sc as plsc

def _dot(a, b, preferred_element_type=jnp.float32):
  return jnp.dot(a, b, preferred_element_type=preferred_element_type)

N = 10000
E = 320000
NS = 16
WIN = 80
NROW = E // WIN
WROWS = NROW // NS
ZBLK = 40
NZB = N // ZBLK
ZITER = (NZB + NS - 1) // NS


def _make_agg(F):
  mesh = plsc.VectorSubcoreMesh(core_axis_name="c", subcore_axis_name="s",
                                num_cores=2, num_subcores=NS)
  params = (None if F == 128 else
            pltpu.CompilerParams(use_tc_tiling_on_sc=False))

  @functools.partial(
      pl.kernel,
      out_type=(jax.ShapeDtypeStruct((N, F), jnp.float32),
                jax.ShapeDtypeStruct((N, F), jnp.float32)),
      mesh=mesh,
      compiler_params=params,
      scratch_types=[
          pltpu.VMEM((8, 2, WIN), jnp.int32),
          pltpu.VMEM((4, WIN, F), jnp.float32),
          pltpu.VMEM_SHARED((N, F), jnp.float32),
          [pltpu.SemaphoreType.DMA] * 4,
          [pltpu.SemaphoreType.DMA] * 4,
          [pltpu.SemaphoreType.DMA] * 8,
      ],
  )
  def agg(hs, hf, ep_s, ep_f, zeros_hbm, out_s, out_f,
          idx, rows, acc, gsems, ssems, isems):
    c = lax.axis_index("c")
    s = lax.axis_index("s")

    def zinit(j, carry):
      b = j * NS + s

      @pl.when(b < NZB)
      def _():
        pltpu.sync_copy(zeros_hbm.at[pl.ds(b * ZBLK, ZBLK)],
                        acc.at[pl.ds(b * ZBLK, ZBLK)])

      return carry

    lax.fori_loop(0, ZITER, zinit, 0)
    plsc.subcore_barrier()

    def run(ep_hbm, tbl_hbm, out_hbm):
      base = s * WROWS

      def fetch(w, q):
        pltpu.async_copy(ep_hbm.at[base + w], idx.at[q], isems[q])

      def wait_i(q):
        pltpu.make_async_copy(ep_hbm.at[base], idx.at[q], isems[q]).wait()

      def gath(w, q, rb):
        del w
        pltpu.async_copy(tbl_hbm.at[idx.at[q, 0]], rows.at[rb], gsems[rb])

      def wait_g(rb):
        pltpu.make_async_copy(tbl_hbm.at[idx.at[0, 0]], rows.at[rb],
                              gsems[rb]).wait()

      def scat(w, q, rb):
        del w
        pltpu.async_copy(rows.at[rb], acc.at[idx.at[q, 1]], ssems[rb],
                         add=True)

      def wait_s(rb):
        pltpu.make_async_copy(rows.at[rb], acc.at[idx.at[0, 1]],
                              ssems[rb]).wait()

      for q in range(6):
        fetch(q, q)
      wait_i(0)
      gath(0, 0, 0)
      wait_i(1)
      gath(1, 1, 1)
      wait_i(2)
      gath(2, 2, 2)
      fetch(6, 6)
      wait_g(0)
      scat(0, 0, 0)
      wait_i(3)
      gath(3, 3, 3)
      fetch(7, 7)
      wait_g(1)
      scat(1, 1, 1)

      def batch(kb, carry):
        for j in range(8):
          w = kb * 8 + j + 2
          r = (j + 2) % 4
          qi = (j + 2) % 8
          wait_s((r + 2) % 4)
          @pl.when(w + 2 < WROWS)
          def _():
            wait_i((qi + 2) % 8)
            gath(w + 2, (qi + 2) % 8, (r + 2) % 4)
          @pl.when(w + 6 < WROWS)
          def _():
            fetch(w + 6, (qi + 6) % 8)
          wait_g(r)
          scat(w, qi, r)
        return carry

      lax.fori_loop(0, (WROWS - 2) // 8, batch, 0)
      wait_s(0)
      wait_s(1)
      plsc.subcore_barrier()

      def zout(j, carry):
        b = j * NS + s

        @pl.when(b < NZB)
        def _():
          pltpu.sync_copy(acc.at[pl.ds(b * ZBLK, ZBLK)],
                          out_hbm.at[pl.ds(b * ZBLK, ZBLK)])

        return carry

      lax.fori_loop(0, ZITER, zout, 0)

    @pl.when(c == 0)
    def _():
      run(ep_s, hs, out_s)

    @pl.when(c == 1)
    def _():
      run(ep_f, hf, out_f)

  return agg


@functools.lru_cache(maxsize=None)
def _get_agg(F):
  return _make_agg(F)


BN = 2000


def _full(shape):
  return pl.BlockSpec(shape, lambda i: (0, 0))


def _rows(cols):
  return pl.BlockSpec((BN, cols), lambda i: (i, 0))


def _k1_body(x_ref, wa_ref, wb_ref, oa_ref, ob_ref):
  xb = x_ref[...]
  oa_ref[...] = _dot(xb, wa_ref[...], preferred_element_type=jnp.float32)
  ob_ref[...] = _dot(xb, wb_ref[...], preferred_element_type=jnp.float32)


def _k1(x, wa, wb):
  return pl.pallas_call(
      _k1_body,
      grid=(N // BN,),
      in_specs=[_rows(128), _full((128, 128)), _full((128, 128))],
      out_specs=(_rows(128), _rows(128)),
      out_shape=(jax.ShapeDtypeStruct((N, 128), jnp.float32),
                 jax.ShapeDtypeStruct((N, 128), jnp.float32)),
  )(x, wa, wb)


def _k2_body(a_ref, b_ref, ba_ref, bb_ref, ws_ref, wf_ref, os_ref, of_ref):
  ha = jnp.maximum(a_ref[...] + ba_ref[...], 0.0)
  hb = jnp.maximum(b_ref[...] + bb_ref[...], 0.0)
  os_ref[...] = _dot(ha, ws_ref[...], preferred_element_type=jnp.float32)
  of_ref[...] = _dot(hb, wf_ref[...], preferred_element_type=jnp.float32)


def _k2(As, Af, b1s, b1f, wblk_s, wblk_f):
  return pl.pallas_call(
      _k2_body,
      grid=(N // BN,),
      in_specs=[_rows(128), _rows(128), _full((1, 128)), _full((1, 128)),
                _full((128, 64)), _full((128, 64))],
      out_specs=(_rows(64), _rows(64)),
      out_shape=(jax.ShapeDtypeStruct((N, 64), jnp.float32),
                 jax.ShapeDtypeStruct((N, 64), jnp.float32)),
  )(As, Af, b1s, b1f, wblk_s, wblk_f)


def _k3_body(bs_ref, bf_ref,
             b_s2_ref, b_c2_ref, b_f2_ref,
             a1_ref, ab1_ref, a2_ref, cvec_ref, wmlp_ref, bmlp_ref,
             wdec_ref, bdec_ref,
             com1_ref, com2_ref, emb_ref, hpre_ref, stats_ref):
  i = pl.program_id(0)
  bs = bs_ref[...]
  bf = bf_ref[...]
  emb1 = bs[:, :32] + b_s2_ref[...]
  com1 = bs[:, 32:] + b_c2_ref[...]
  com2 = bf[:, :32] + b_c2_ref[...]
  emb2 = bf[:, 32:] + b_f2_ref[...]
  com1_ref[...] = com1
  com2_ref[...] = com2
  comavg = (com1 + com2) * 0.5

  a1 = a1_ref[...]
  ab1 = ab1_ref[...]
  a2 = a2_ref[...]

  def att(zi):
    t = jnp.tanh(_dot(zi, a1, preferred_element_type=jnp.float32) + ab1)
    return _dot(t, a2, preferred_element_type=jnp.float32)

  w1 = att(emb1)
  w2 = att(comavg)
  w3 = att(emb2)
  m = jnp.maximum(jnp.maximum(w1, w2), w3)
  e1 = jnp.exp(w1 - m)
  e2 = jnp.exp(w2 - m)
  e3 = jnp.exp(w3 - m)
  inv = 1.0 / (e1 + e2 + e3)
  emb_att = (e1 * emb1 + e2 * comavg + e3 * emb2) * inv

  cv = jnp.tanh(cvec_ref[...])
  emb1c = cv[0, 0] * emb1 + cv[0, 1] * emb2 + cv[0, 2] * comavg

  emb = _dot(emb1c + emb_att, wmlp_ref[...],
                preferred_element_type=jnp.float32) + bmlp_ref[...]
  emb_ref[...] = emb

  hpre = _dot(emb, wdec_ref[...],
                 preferred_element_type=jnp.float32) + bdec_ref[...]
  hpre_ref[...] = hpre

  s1 = jnp.sum(hpre, axis=0, keepdims=True)
  s2 = jnp.sum(hpre * hpre, axis=0, keepdims=True)
  block = jnp.concatenate([s1, s2, jnp.zeros((6, 64), jnp.float32)], axis=0)

  @pl.when(i == 0)
  def _():
    stats_ref[...] = jnp.zeros_like(stats_ref)

  stats_ref[...] += block


def _k3(AggS, AggF, b_s2, b_c2, b_f2, A1, ab1, a2col, cvec,
        W_mlp, b_mlp, W_dec, b_dec):
  return pl.pallas_call(
      _k3_body,
      grid=(N // BN,),
      in_specs=[_rows(64), _rows(64),
                _full((1, 32)), _full((1, 32)),
                _full((1, 32)), _full((32, 16)), _full((1, 16)),
                _full((16, 1)), _full((1, 3)), _full((32, 32)),
                _full((1, 32)), _full((32, 64)), _full((1, 64))],
      out_specs=(_rows(32), _rows(32), _rows(32), _rows(64),
                 pl.BlockSpec((8, 64), lambda i: (0, 0))),
      out_shape=(jax.ShapeDtypeStruct((N, 32), jnp.float32),
                 jax.ShapeDtypeStruct((N, 32), jnp.float32),
                 jax.ShapeDtypeStruct((N, 32), jnp.float32),
                 jax.ShapeDtypeStruct((N, 64), jnp.float32),
                 jax.ShapeDtypeStruct((8, 64), jnp.float32)),
  )(AggS, AggF, b_s2, b_c2, b_f2, A1, ab1, a2col, cvec,
    W_mlp, b_mlp, W_dec, b_dec)


def _k4_body(hpre_ref, stats_ref, bng_ref, bnb_ref,
             wpi_ref, bpi_ref, wdisp_ref, bdisp_ref, wmean_ref, bmean_ref,
             pi_ref, disp_ref, mean_ref):
  stats = stats_ref[...]
  mu = stats[0:1, :] * (1.0 / N)
  ex2 = stats[1:2, :] * (1.0 / N)
  var = ex2 - mu * mu
  scale = bng_ref[...] * jax.lax.rsqrt(var + 1e-5)
  h = (hpre_ref[...] - mu) * scale + bnb_ref[...]
  h = jnp.maximum(h, 0.0)
  pi_ref[...] = jax.nn.sigmoid(
      _dot(h, wpi_ref[...], preferred_element_type=jnp.float32)
      + bpi_ref[...])
  disp_ref[...] = jnp.clip(
      jax.nn.softplus(_dot(h, wdisp_ref[...],
                              preferred_element_type=jnp.float32)
                      + bdisp_ref[...]), 1e-4, 1e4)
  mean_ref[...] = jnp.clip(
      jnp.exp(_dot(h, wmean_ref[...], preferred_element_type=jnp.float32)
              + bmean_ref[...]), 1e-5, 1e6)


def _k4(hpre, stats, bn_g, bn_b, W_pi, b_pi, W_disp, b_disp, W_mean, b_mean):
  return pl.pallas_call(
      _k4_body,
      grid=(N // BN,),
      in_specs=[_rows(64), pl.BlockSpec((8, 64), lambda i: (0, 0)),
                _full((1, 64)), _full((1, 64)),
                _full((64, 128)), _full((1, 128)),
                _full((64, 128)), _full((1, 128)),
                _full((64, 128)), _full((1, 128))],
      out_specs=(_rows(128), _rows(128), _rows(128)),
      out_shape=(jax.ShapeDtypeStruct((N, 128), jnp.float32),
                 jax.ShapeDtypeStruct((N, 128), jnp.float32),
                 jax.ShapeDtypeStruct((N, 128), jnp.float32)),
  )(hpre, stats, bn_g, bn_b, W_pi, b_pi, W_disp, b_disp, W_mean, b_mean)


def kernel(x, sadj, fadj, W_s1, b_s1, W_s2, b_s2, W_f1, b_f1, W_f2, b_f2,
           W_c1, b_c1, W_c2, b_c2, A1, ab1, A2, W_mlp, b_mlp, c_f, c_s,
           c_com, W_dec, b_dec, bn_g, bn_b, W_pi, b_pi, W_disp, b_disp,
           W_mean, b_mean):
  f32 = jnp.float32
  Wsc = jnp.concatenate([W_s1, W_c1], axis=1)
  Wcf = jnp.concatenate([W_c1, W_f1], axis=1)
  b1s = jnp.concatenate([b_s1, b_c1])[None, :]
  b1f = jnp.concatenate([b_c1, b_f1])[None, :]
  Z = jnp.zeros((64, 32), f32)
  wblk_s = jnp.concatenate(
      [jnp.concatenate([W_s2, Z], axis=1),
       jnp.concatenate([Z, W_c2], axis=1)], axis=0)
  wblk_f = jnp.concatenate(
      [jnp.concatenate([W_c2, Z], axis=1),
       jnp.concatenate([Z, W_f2], axis=1)], axis=0)
  cvec = jnp.stack([c_f, c_s, c_com]).reshape(1, 3)
  ep_s = jnp.stack([sadj[0].reshape(NROW, WIN),
                    sadj[1].reshape(NROW, WIN)], axis=1)
  ep_f = jnp.stack([fadj[0].reshape(NROW, WIN),
                    fadj[1].reshape(NROW, WIN)], axis=1)
  zeros128 = jnp.zeros((N, 128), f32)

  hs, hf = _k1(x, Wsc, Wcf)
  As, Af = _get_agg(128)(hs, hf, ep_s, ep_f, zeros128)

  g_s, g_f = _k2(As, Af, b1s, b1f, wblk_s, wblk_f)
  zeros64 = jnp.zeros((N, 64), f32)
  AggS, AggF = _get_agg(64)(g_s, g_f, ep_s, ep_f, zeros64)

  com1, com2, emb, hpre, stats = _k3(
      AggS, AggF, b_s2[None, :], b_c2[None, :],
      b_f2[None, :], A1, ab1[None, :], A2, cvec, W_mlp,
      b_mlp[None, :], W_dec, b_dec[None, :])
  pi, disp, mean = _k4(hpre, stats, bn_g[None, :], bn_b[None, :],
                       W_pi, b_pi[None, :], W_disp, b_disp[None, :],
                       W_mean, b_mean[None, :])
  return (com1, com2, emb, pi, disp, mean)


# speedup vs baseline: 15.3835x; 1.0698x over previous
import functools

import jax
import jax.numpy as jnp
from jax import lax
from jax.experimental import pallas as pl
from jax.experimental.pallas import tpu as pltpu
from jax.experimental.pallas import tpu_sc as plsc

def _dot(a, b, preferred_element_type=jnp.float32):
  return jnp.dot(a, b, preferred_element_type=preferred_element_type)

N = 10000
E = 320000
NS = 16
WIN = 80
NROW = E // WIN
WROWS = NROW // NS
ZBLK = 40
NZB = N // ZBLK
ZITER = (NZB + NS - 1) // NS


def _make_agg(F):
  mesh = plsc.VectorSubcoreMesh(core_axis_name="c", subcore_axis_name="s",
                                num_cores=2, num_subcores=NS)
  params = (None if F == 128 else
            pltpu.CompilerParams(use_tc_tiling_on_sc=False))

  @functools.partial(
      pl.kernel,
      out_type=(jax.ShapeDtypeStruct((N, F), jnp.float32),
                jax.ShapeDtypeStruct((N, F), jnp.float32)),
      mesh=mesh,
      compiler_params=params,
      scratch_types=[
          pltpu.VMEM((8, 2, WIN), jnp.int32),
          pltpu.VMEM((4, WIN, F), jnp.float32),
          pltpu.VMEM_SHARED((N, F), jnp.float32),
          [pltpu.SemaphoreType.DMA] * 4,
          [pltpu.SemaphoreType.DMA] * 4,
          [pltpu.SemaphoreType.DMA] * 8,
      ],
  )
  def agg(hs, hf, ep_s, ep_f, zeros_hbm, out_s, out_f,
          idx, rows, acc, gsems, ssems, isems):
    c = lax.axis_index("c")
    s = lax.axis_index("s")

    def run(ep_hbm, tbl_hbm, out_hbm):
      base = s * WROWS

      def fetch(w, q):
        pltpu.async_copy(ep_hbm.at[base + w], idx.at[q], isems[q])

      def wait_i(q):
        pltpu.make_async_copy(ep_hbm.at[base], idx.at[q], isems[q]).wait()

      def gath(w, q, rb):
        del w
        pltpu.async_copy(tbl_hbm.at[idx.at[q, 0]], rows.at[rb], gsems[rb])

      def wait_g(rb):
        pltpu.make_async_copy(tbl_hbm.at[idx.at[0, 0]], rows.at[rb],
                              gsems[rb]).wait()

      def scat(w, q, rb):
        del w
        pltpu.async_copy(rows.at[rb], acc.at[idx.at[q, 1]], ssems[rb],
                         add=True)

      def wait_s(rb):
        pltpu.make_async_copy(rows.at[rb], acc.at[idx.at[0, 1]],
                              ssems[rb]).wait()

      def zinit(j, carry):
        b = j * NS + s

        @pl.when(b < NZB)
        def _():
          pltpu.async_copy(zeros_hbm.at[pl.ds(b * ZBLK, ZBLK)],
                           acc.at[pl.ds(b * ZBLK, ZBLK)], ssems[3])

        return carry

      lax.fori_loop(0, ZITER, zinit, 0)

      for q in range(6):
        fetch(q, q)
      wait_i(0)
      gath(0, 0, 0)
      wait_i(1)
      gath(1, 1, 1)

      def zdrain(j, carry):
        b = j * NS + s

        @pl.when(b < NZB)
        def _():
          pltpu.make_async_copy(zeros_hbm.at[pl.ds(0, ZBLK)],
                                acc.at[pl.ds(0, ZBLK)], ssems[3]).wait()

        return carry

      lax.fori_loop(0, ZITER, zdrain, 0)
      plsc.subcore_barrier()

      wait_i(2)
      gath(2, 2, 2)
      fetch(6, 6)
      wait_g(0)
      scat(0, 0, 0)
      wait_i(3)
      gath(3, 3, 3)
      fetch(7, 7)
      wait_g(1)
      scat(1, 1, 1)

      def batch(kb, carry):
        for j in range(8):
          w = kb * 8 + j + 2
          r = (j + 2) % 4
          qi = (j + 2) % 8
          wait_s((r + 2) % 4)
          @pl.when(w + 2 < WROWS)
          def _():
            wait_i((qi + 2) % 8)
            gath(w + 2, (qi + 2) % 8, (r + 2) % 4)
          @pl.when(w + 6 < WROWS)
          def _():
            fetch(w + 6, (qi + 6) % 8)
          wait_g(r)
          scat(w, qi, r)
        return carry

      lax.fori_loop(0, (WROWS - 2) // 8, batch, 0)
      wait_s(0)
      wait_s(1)
      plsc.subcore_barrier()

      def zout(j, carry):
        b = j * NS + s

        @pl.when(b < NZB)
        def _():
          pltpu.async_copy(acc.at[pl.ds(b * ZBLK, ZBLK)],
                           out_hbm.at[pl.ds(b * ZBLK, ZBLK)], ssems[3])

        return carry

      lax.fori_loop(0, ZITER, zout, 0)

      def zout_drain(j, carry):
        b = j * NS + s

        @pl.when(b < NZB)
        def _():
          pltpu.make_async_copy(acc.at[pl.ds(0, ZBLK)],
                                out_hbm.at[pl.ds(0, ZBLK)],
                                ssems[3]).wait()

        return carry

      lax.fori_loop(0, ZITER, zout_drain, 0)

    @pl.when(c == 0)
    def _():
      run(ep_s, hs, out_s)

    @pl.when(c == 1)
    def _():
      run(ep_f, hf, out_f)

  return agg


@functools.lru_cache(maxsize=None)
def _get_agg(F):
  return _make_agg(F)


BN = 2000


def _full(shape):
  return pl.BlockSpec(shape, lambda i: (0, 0))


def _rows(cols):
  return pl.BlockSpec((BN, cols), lambda i: (i, 0))


def _k1_body(x_ref, wa_ref, wb_ref, oa_ref, ob_ref):
  xb = x_ref[...]
  oa_ref[...] = _dot(xb, wa_ref[...], preferred_element_type=jnp.float32)
  ob_ref[...] = _dot(xb, wb_ref[...], preferred_element_type=jnp.float32)


def _k1(x, wa, wb):
  return pl.pallas_call(
      _k1_body,
      grid=(N // BN,),
      in_specs=[_rows(128), _full((128, 128)), _full((128, 128))],
      out_specs=(_rows(128), _rows(128)),
      out_shape=(jax.ShapeDtypeStruct((N, 128), jnp.float32),
                 jax.ShapeDtypeStruct((N, 128), jnp.float32)),
  )(x, wa, wb)


def _k2_body(a_ref, b_ref, ba_ref, bb_ref, ws_ref, wf_ref, os_ref, of_ref):
  ha = jnp.maximum(a_ref[...] + ba_ref[...], 0.0)
  hb = jnp.maximum(b_ref[...] + bb_ref[...], 0.0)
  os_ref[...] = _dot(ha, ws_ref[...], preferred_element_type=jnp.float32)
  of_ref[...] = _dot(hb, wf_ref[...], preferred_element_type=jnp.float32)


def _k2(As, Af, b1s, b1f, wblk_s, wblk_f):
  return pl.pallas_call(
      _k2_body,
      grid=(N // BN,),
      in_specs=[_rows(128), _rows(128), _full((1, 128)), _full((1, 128)),
                _full((128, 64)), _full((128, 64))],
      out_specs=(_rows(64), _rows(64)),
      out_shape=(jax.ShapeDtypeStruct((N, 64), jnp.float32),
                 jax.ShapeDtypeStruct((N, 64), jnp.float32)),
  )(As, Af, b1s, b1f, wblk_s, wblk_f)


def _k3_body(bs_ref, bf_ref,
             b_s2_ref, b_c2_ref, b_f2_ref,
             a1_ref, ab1_ref, a2_ref, cvec_ref, wmlp_ref, bmlp_ref,
             wdec_ref, bdec_ref,
             com1_ref, com2_ref, emb_ref, hpre_ref, stats_ref):
  i = pl.program_id(0)
  bs = bs_ref[...]
  bf = bf_ref[...]
  emb1 = bs[:, :32] + b_s2_ref[...]
  com1 = bs[:, 32:] + b_c2_ref[...]
  com2 = bf[:, :32] + b_c2_ref[...]
  emb2 = bf[:, 32:] + b_f2_ref[...]
  com1_ref[...] = com1
  com2_ref[...] = com2
  comavg = (com1 + com2) * 0.5

  a1 = a1_ref[...]
  ab1 = ab1_ref[...]
  a2 = a2_ref[...]

  def att(zi):
    t = jnp.tanh(_dot(zi, a1, preferred_element_type=jnp.float32) + ab1)
    return _dot(t, a2, preferred_element_type=jnp.float32)

  w1 = att(emb1)
  w2 = att(comavg)
  w3 = att(emb2)
  m = jnp.maximum(jnp.maximum(w1, w2), w3)
  e1 = jnp.exp(w1 - m)
  e2 = jnp.exp(w2 - m)
  e3 = jnp.exp(w3 - m)
  inv = 1.0 / (e1 + e2 + e3)
  emb_att = (e1 * emb1 + e2 * comavg + e3 * emb2) * inv

  cv = jnp.tanh(cvec_ref[...])
  emb1c = cv[0, 0] * emb1 + cv[0, 1] * emb2 + cv[0, 2] * comavg

  emb = _dot(emb1c + emb_att, wmlp_ref[...],
                preferred_element_type=jnp.float32) + bmlp_ref[...]
  emb_ref[...] = emb

  hpre = _dot(emb, wdec_ref[...],
                 preferred_element_type=jnp.float32) + bdec_ref[...]
  hpre_ref[...] = hpre

  s1 = jnp.sum(hpre, axis=0, keepdims=True)
  s2 = jnp.sum(hpre * hpre, axis=0, keepdims=True)
  block = jnp.concatenate([s1, s2, jnp.zeros((6, 64), jnp.float32)], axis=0)

  @pl.when(i == 0)
  def _():
    stats_ref[...] = jnp.zeros_like(stats_ref)

  stats_ref[...] += block


def _k3(AggS, AggF, b_s2, b_c2, b_f2, A1, ab1, a2col, cvec,
        W_mlp, b_mlp, W_dec, b_dec):
  return pl.pallas_call(
      _k3_body,
      grid=(N // BN,),
      in_specs=[_rows(64), _rows(64),
                _full((1, 32)), _full((1, 32)),
                _full((1, 32)), _full((32, 16)), _full((1, 16)),
                _full((16, 1)), _full((1, 3)), _full((32, 32)),
                _full((1, 32)), _full((32, 64)), _full((1, 64))],
      out_specs=(_rows(32), _rows(32), _rows(32), _rows(64),
                 pl.BlockSpec((8, 64), lambda i: (0, 0))),
      out_shape=(jax.ShapeDtypeStruct((N, 32), jnp.float32),
                 jax.ShapeDtypeStruct((N, 32), jnp.float32),
                 jax.ShapeDtypeStruct((N, 32), jnp.float32),
                 jax.ShapeDtypeStruct((N, 64), jnp.float32),
                 jax.ShapeDtypeStruct((8, 64), jnp.float32)),
  )(AggS, AggF, b_s2, b_c2, b_f2, A1, ab1, a2col, cvec,
    W_mlp, b_mlp, W_dec, b_dec)


def _k4_body(hpre_ref, stats_ref, bng_ref, bnb_ref,
             wpi_ref, bpi_ref, wdisp_ref, bdisp_ref, wmean_ref, bmean_ref,
             pi_ref, disp_ref, mean_ref):
  stats = stats_ref[...]
  mu = stats[0:1, :] * (1.0 / N)
  ex2 = stats[1:2, :] * (1.0 / N)
  var = ex2 - mu * mu
  scale = bng_ref[...] * jax.lax.rsqrt(var + 1e-5)
  h = (hpre_ref[...] - mu) * scale + bnb_ref[...]
  h = jnp.maximum(h, 0.0)
  pi_ref[...] = jax.nn.sigmoid(
      _dot(h, wpi_ref[...], preferred_element_type=jnp.float32)
      + bpi_ref[...])
  disp_ref[...] = jnp.clip(
      jax.nn.softplus(_dot(h, wdisp_ref[...],
                              preferred_element_type=jnp.float32)
                      + bdisp_ref[...]), 1e-4, 1e4)
  mean_ref[...] = jnp.clip(
      jnp.exp(_dot(h, wmean_ref[...], preferred_element_type=jnp.float32)
              + bmean_ref[...]), 1e-5, 1e6)


def _k4(hpre, stats, bn_g, bn_b, W_pi, b_pi, W_disp, b_disp, W_mean, b_mean):
  return pl.pallas_call(
      _k4_body,
      grid=(N // BN,),
      in_specs=[_rows(64), pl.BlockSpec((8, 64), lambda i: (0, 0)),
                _full((1, 64)), _full((1, 64)),
                _full((64, 128)), _full((1, 128)),
                _full((64, 128)), _full((1, 128)),
                _full((64, 128)), _full((1, 128))],
      out_specs=(_rows(128), _rows(128), _rows(128)),
      out_shape=(jax.ShapeDtypeStruct((N, 128), jnp.float32),
                 jax.ShapeDtypeStruct((N, 128), jnp.float32),
                 jax.ShapeDtypeStruct((N, 128), jnp.float32)),
  )(hpre, stats, bn_g, bn_b, W_pi, b_pi, W_disp, b_disp, W_mean, b_mean)


def kernel(x, sadj, fadj, W_s1, b_s1, W_s2, b_s2, W_f1, b_f1, W_f2, b_f2,
           W_c1, b_c1, W_c2, b_c2, A1, ab1, A2, W_mlp, b_mlp, c_f, c_s,
           c_com, W_dec, b_dec, bn_g, bn_b, W_pi, b_pi, W_disp, b_disp,
           W_mean, b_mean):
  f32 = jnp.float32
  Wsc = jnp.concatenate([W_s1, W_c1], axis=1)
  Wcf = jnp.concatenate([W_c1, W_f1], axis=1)
  b1s = jnp.concatenate([b_s1, b_c1])[None, :]
  b1f = jnp.concatenate([b_c1, b_f1])[None, :]
  Z = jnp.zeros((64, 32), f32)
  wblk_s = jnp.concatenate(
      [jnp.concatenate([W_s2, Z], axis=1),
       jnp.concatenate([Z, W_c2], axis=1)], axis=0)
  wblk_f = jnp.concatenate(
      [jnp.concatenate([W_c2, Z], axis=1),
       jnp.concatenate([Z, W_f2], axis=1)], axis=0)
  cvec = jnp.stack([c_f, c_s, c_com]).reshape(1, 3)
  ep_s = jnp.stack([sadj[0].reshape(NROW, WIN),
                    sadj[1].reshape(NROW, WIN)], axis=1)
  ep_f = jnp.stack([fadj[0].reshape(NROW, WIN),
                    fadj[1].reshape(NROW, WIN)], axis=1)
  zeros128 = jnp.zeros((N, 128), f32)

  hs, hf = _k1(x, Wsc, Wcf)
  As, Af = _get_agg(128)(hs, hf, ep_s, ep_f, zeros128)

  g_s, g_f = _k2(As, Af, b1s, b1f, wblk_s, wblk_f)
  zeros64 = jnp.zeros((N, 64), f32)
  AggS, AggF = _get_agg(64)(g_s, g_f, ep_s, ep_f, zeros64)

  com1, com2, emb, hpre, stats = _k3(
      AggS, AggF, b_s2[None, :], b_c2[None, :],
      b_f2[None, :], A1, ab1[None, :], A2, cvec, W_mlp,
      b_mlp[None, :], W_dec, b_dec[None, :])
  pi, disp, mean = _k4(hpre, stats, bn_g[None, :], bn_b[None, :],
                       W_pi, b_pi[None, :], W_disp, b_disp[None, :],
                       W_mean, b_mean[None, :])
  return (com1, com2, emb, pi, disp, mean)
